# Initial kernel scaffold; baseline (speedup 1.0000x reference)
import jax
import jax.numpy as jnp
from jax.experimental import pallas as pl


def kernel(route_emb, route_len, adj_matrices, W0, a0, g0, b0, W1, a1, g1, b1):
    raise NotImplementedError("write your pallas kernel here")


# speedup vs baseline: 37.8636x; 37.8636x over previous
import functools

import jax
import jax.numpy as jnp
from jax import lax
from jax.experimental import pallas as pl
from jax.experimental.pallas import tpu as pltpu
from jax.experimental.pallas import tpu_sc as plsc

L, B, H, E, NH = 4096, 8, 128, 16384, 4
HD = H // NH
A2 = 2 * NH
SES = 128

NC = 2
NT = 16
GPC = B // NC
EPT = E // NT
CH = 64
NCHUNK = EPT // CH
RPT = L // NT
ZR = 32

_RT = 512


def _build_a_mat(a):
    eye = jnp.eye(NH, dtype=jnp.float32)
    asrc = jnp.einsum('hd,hk->hdk', a[:, :HD], eye).reshape(H, NH)
    adst = jnp.einsum('hd,hk->hdk', a[:, HD:], eye).reshape(H, NH)
    return jnp.concatenate([asrc, adst], axis=1)


def _dense_body(x_ref, w_ref, am_ref, h_ref, al_ref):
    h = jnp.dot(x_ref[0], w_ref[...], preferred_element_type=jnp.float32)
    h_ref[0] = h
    al_ref[0] = jnp.dot(h, am_ref[...], preferred_element_type=jnp.float32)


def _dense(x, w, am):
    return pl.pallas_call(
        _dense_body,
        grid=(B, L // _RT),
        in_specs=[
            pl.BlockSpec((1, _RT, H), lambda b, t: (b, t, 0)),
            pl.BlockSpec((H, H), lambda b, t: (0, 0)),
            pl.BlockSpec((H, A2), lambda b, t: (0, 0)),
        ],
        out_specs=[
            pl.BlockSpec((1, _RT, H), lambda b, t: (b, t, 0)),
            pl.BlockSpec((1, _RT, A2), lambda b, t: (b, t, 0)),
        ],
        out_shape=[
            jax.ShapeDtypeStruct((B, L, H), jnp.float32),
            jax.ShapeDtypeStruct((B, L, A2), jnp.float32),
        ],
    )(x, w, am)


def _epilogue(x, acc, ses, sel, g, bb):
    recip = 1.0 / (ses[:, :NH] + 1e-16)
    mult = jnp.dot(recip, sel, preferred_element_type=jnp.float32)
    y = x + acc * mult
    m = jnp.mean(y, axis=-1, keepdims=True)
    v = jnp.mean((y - m) ** 2, axis=-1, keepdims=True)
    y = (y - m) / jnp.sqrt(v + 1e-5) * g + bb
    return jnp.maximum(y, 0.0)


def _post_dense_body(x_ref, acc_ref, ses_ref, sel_ref, g_ref, b_ref,
                     w_ref, am_ref, y_ref, h_ref, al_ref):
    y = _epilogue(x_ref[0], acc_ref[0], ses_ref[0], sel_ref[...],
                  g_ref[...], b_ref[...])
    y_ref[0] = y
    h = jnp.dot(y, w_ref[...], preferred_element_type=jnp.float32)
    h_ref[0] = h
    al_ref[0] = jnp.dot(h, am_ref[...], preferred_element_type=jnp.float32)


def _post_dense(x, acc, ses, sel, g, bb, w, am):
    return pl.pallas_call(
        _post_dense_body,
        grid=(B, L // _RT),
        in_specs=[
            pl.BlockSpec((1, _RT, H), lambda b, t: (b, t, 0)),
            pl.BlockSpec((1, _RT, H), lambda b, t: (b, t, 0)),
            pl.BlockSpec((1, _RT, SES), lambda b, t: (b, t, 0)),
            pl.BlockSpec((NH, H), lambda b, t: (0, 0)),
            pl.BlockSpec((1, H), lambda b, t: (0, 0)),
            pl.BlockSpec((1, H), lambda b, t: (0, 0)),
            pl.BlockSpec((H, H), lambda b, t: (0, 0)),
            pl.BlockSpec((H, A2), lambda b, t: (0, 0)),
        ],
        out_specs=[
            pl.BlockSpec((1, _RT, H), lambda b, t: (b, t, 0)),
            pl.BlockSpec((1, _RT, H), lambda b, t: (b, t, 0)),
            pl.BlockSpec((1, _RT, A2), lambda b, t: (b, t, 0)),
        ],
        out_shape=[
            jax.ShapeDtypeStruct((B, L, H), jnp.float32),
            jax.ShapeDtypeStruct((B, L, H), jnp.float32),
            jax.ShapeDtypeStruct((B, L, A2), jnp.float32),
        ],
    )(x, acc, ses, sel, g, bb, w, am)


def _post_mask_body(len_ref, x_ref, acc_ref, ses_ref, sel_ref, g_ref, b_ref,
                    o_ref):
    y = _epilogue(x_ref[0], acc_ref[0], ses_ref[0], sel_ref[...],
                  g_ref[...], b_ref[...])
    vl = len_ref[pl.program_id(0)]
    rows = (lax.broadcasted_iota(jnp.int32, (_RT, 1), 0)
            + pl.program_id(1) * _RT)
    o_ref[0] = jnp.where(rows < vl, y, 0.0)


def _post_mask(route_len, x, acc, ses, sel, g, bb):
    return pl.pallas_call(
        _post_mask_body,
        grid=(B, L // _RT),
        in_specs=[
            pl.BlockSpec(memory_space=pltpu.SMEM),
            pl.BlockSpec((1, _RT, H), lambda b, t: (b, t, 0)),
            pl.BlockSpec((1, _RT, H), lambda b, t: (b, t, 0)),
            pl.BlockSpec((1, _RT, SES), lambda b, t: (b, t, 0)),
            pl.BlockSpec((NH, H), lambda b, t: (0, 0)),
            pl.BlockSpec((1, H), lambda b, t: (0, 0)),
            pl.BlockSpec((1, H), lambda b, t: (0, 0)),
        ],
        out_specs=pl.BlockSpec((1, _RT, H), lambda b, t: (b, t, 0)),
        out_shape=jax.ShapeDtypeStruct((B, L, H), jnp.float32),
    )(route_len, x, acc, ses, sel, g, bb)


_MESH = plsc.VectorSubcoreMesh(core_axis_name="c", subcore_axis_name="s",
                               num_cores=NC)


@functools.partial(
    pl.kernel,
    mesh=_MESH,
    out_type=[
        jax.ShapeDtypeStruct((B * L, H), jnp.float32),
        jax.ShapeDtypeStruct((B * L, SES), jnp.float32),
    ],
    scratch_types=[
        pltpu.VMEM_SHARED((L, H), jnp.float32),
        pltpu.VMEM((L * A2,), jnp.float32),
        pltpu.VMEM((ZR, H), jnp.float32),
        pltpu.VMEM((CH, H), jnp.float32),
        pltpu.VMEM((CH, H), jnp.float32),
        pltpu.VMEM((EPT * NH,), jnp.float32),
        pltpu.VMEM((CH,), jnp.int32),
        pltpu.VMEM((CH,), jnp.int32),
        pltpu.VMEM((CH,), jnp.int32),
        pltpu.SemaphoreType.DMA,
    ],
    compiler_params=pltpu.CompilerParams(needs_layout_passes=False),
)
def _edge_kernel(h_hbm, al_hbm, ei_hbm, acc_hbm, ses_hbm,
                 acc_sp, alpha_buf, zrows, rows_buf, esrow_buf,
                 es_all, src_buf, dst_buf, gsrc_buf, sem):
    c = lax.axis_index("c")
    s = lax.axis_index("s")
    r0 = s * RPT
    zero16 = jnp.zeros((16,), jnp.float32)

    def _zrow(r, carry):
        for j in range(H // 16):
            zrows[r, pl.ds(j * 16, 16)] = zero16
        return carry
    lax.fori_loop(0, ZR, _zrow, 0)

    def _zesr(r, carry):
        for j in range(H // 16):
            esrow_buf[r, pl.ds(j * 16, 16)] = zero16
        return carry
    lax.fori_loop(0, CH, _zesr, 0)

    lane = lax.iota(jnp.int32, 16)
    lanehC = (lane & (NH - 1)) * CH
    mask4 = jnp.where(lane < NH, 1.0, 0.0).astype(jnp.float32)

    def _zero_acc():
        for z in range(RPT // ZR):
            pltpu.sync_copy(zrows, acc_sp.at[pl.ds(r0 + z * ZR, ZR)])

    def _graph(i, carry):
        b = c * GPC + i
        base = b * L
        pltpu.sync_copy(al_hbm.at[pl.ds(base * A2, L * A2)], alpha_buf)
        _zero_acc()
        plsc.subcore_barrier()

        def _chunk(j, carry2):
            e0 = s * EPT + j * CH
            pltpu.sync_copy(ei_hbm.at[2 * b, pl.ds(e0, CH)], src_buf)
            pltpu.sync_copy(ei_hbm.at[2 * b + 1, pl.ds(e0, CH)], dst_buf)
            ebase = j * NH * CH
            for g in range(CH // 16):
                sv = src_buf[pl.ds(g * 16, 16)]
                dv = dst_buf[pl.ds(g * 16, 16)]
                gsrc_buf[pl.ds(g * 16, 16)] = sv + base
                s8 = sv * A2
                d8 = dv * A2
                for h in range(NH):
                    a_s = plsc.load_gather(alpha_buf, [s8 + h])
                    a_d = plsc.load_gather(alpha_buf, [d8 + (NH + h)])
                    es_all[pl.ds(ebase + h * CH + g * 16, 16)] = a_s + a_d
            for q in range(NH * CH // 16):
                sc = es_all[pl.ds(ebase + q * 16, 16)]
                es_all[pl.ds(ebase + q * 16, 16)] = jnp.exp(
                    jnp.maximum(sc, 0.2 * sc))
            pltpu.async_copy(h_hbm.at[gsrc_buf], rows_buf, sem).wait()

            def _scale(k, carry3):
                eb = [plsc.load_gather(es_all,
                                       [jnp.full((16,), ebase + h * CH + k,
                                                 jnp.int32)])
                      for h in range(NH)]
                for j2 in range(H // 16):
                    rows_buf[k, pl.ds(j2 * 16, 16)] = (
                        rows_buf[k, pl.ds(j2 * 16, 16)] * eb[j2 // 2])
                return carry3
            lax.fori_loop(0, CH, _scale, 0)
            pltpu.sync_copy(rows_buf, acc_sp.at[dst_buf], add=True)
            return carry2
        lax.fori_loop(0, NCHUNK, _chunk, 0)
        plsc.subcore_barrier()
        pltpu.sync_copy(acc_sp.at[pl.ds(r0, RPT)],
                        acc_hbm.at[pl.ds(base + r0, RPT)])
        _zero_acc()
        plsc.subcore_barrier()

        def _chunk2(j, carry2):
            e0 = s * EPT + j * CH
            pltpu.sync_copy(ei_hbm.at[2 * b + 1, pl.ds(e0, CH)], dst_buf)

            def _fill(k, carry3):
                esr = plsc.load_gather(es_all, [lanehC + (k + j * NH * CH)])
                esrow_buf[k, pl.ds(0, 16)] = esr * mask4
                return carry3
            lax.fori_loop(0, CH, _fill, 0)
            pltpu.sync_copy(esrow_buf, acc_sp.at[dst_buf], add=True)
            return carry2
        lax.fori_loop(0, NCHUNK, _chunk2, 0)
        plsc.subcore_barrier()
        pltpu.sync_copy(acc_sp.at[pl.ds(r0, RPT)],
                        ses_hbm.at[pl.ds(base + r0, RPT)])
        _zero_acc()
        plsc.subcore_barrier()
        return carry
    lax.fori_loop(0, GPC, _graph, 0)


def kernel(route_emb, route_len, adj_matrices, W0, a0, g0, b0, W1, a1, g1, b1):
    x0 = jnp.transpose(route_emb, (1, 0, 2))
    ei = adj_matrices.astype(jnp.int32).reshape(B * 2, E)
    am0 = _build_a_mat(a0)
    am1 = _build_a_mat(a1)
    sel = jnp.repeat(jnp.eye(NH, dtype=jnp.float32), HD, axis=1)
    g0r, b0r = g0.reshape(1, H), b0.reshape(1, H)
    g1r, b1r = g1.reshape(1, H), b1.reshape(1, H)

    h0, al0 = _dense(x0, W0, am0)
    acc0, ses0 = _edge_kernel(h0.reshape(B * L, H), al0.reshape(B * L * A2), ei)
    y1, h1, al1 = _post_dense(x0, acc0.reshape(B, L, H),
                              ses0.reshape(B, L, SES), sel, g0r, b0r, W1, am1)
    acc1, ses1 = _edge_kernel(h1.reshape(B * L, H), al1.reshape(B * L * A2), ei)
    out = _post_mask(route_len.astype(jnp.int32), y1, acc1.reshape(B, L, H),
                     ses1.reshape(B, L, SES), sel, g1r, b1r)
    return jnp.transpose(out, (1, 0, 2))


# speedup vs baseline: 39.2997x; 1.0379x over previous
import functools

import jax
import jax.numpy as jnp
from jax import lax
from jax.experimental import pallas as pl
from jax.experimental.pallas import tpu as pltpu
from jax.experimental.pallas import tpu_sc as plsc

L, B, H, E, NH = 4096, 8, 128, 16384, 4
HD = H // NH
A2 = 2 * NH
SES = 128

NC = 2
NT = 16
GPC = B // NC
EPT = E // NT
CH = 64
NCHUNK = EPT // CH
RPT = L // NT
ZR = 32

_RT = 512


def _build_a_mat(a):
    eye = jnp.eye(NH, dtype=jnp.float32)
    asrc = jnp.einsum('hd,hk->hdk', a[:, :HD], eye).reshape(H, NH)
    adst = jnp.einsum('hd,hk->hdk', a[:, HD:], eye).reshape(H, NH)
    return jnp.concatenate([asrc, adst], axis=1)


def _dense_body(x_ref, w_ref, am_ref, h_ref, al_ref):
    h = jnp.dot(x_ref[0], w_ref[...], preferred_element_type=jnp.float32)
    h_ref[0] = h
    al_ref[0] = jnp.dot(h, am_ref[...], preferred_element_type=jnp.float32)


def _dense(x, w, am):
    return pl.pallas_call(
        _dense_body,
        grid=(B, L // _RT),
        in_specs=[
            pl.BlockSpec((1, _RT, H), lambda b, t: (b, t, 0)),
            pl.BlockSpec((H, H), lambda b, t: (0, 0)),
            pl.BlockSpec((H, A2), lambda b, t: (0, 0)),
        ],
        out_specs=[
            pl.BlockSpec((1, _RT, H), lambda b, t: (b, t, 0)),
            pl.BlockSpec((1, _RT, A2), lambda b, t: (b, t, 0)),
        ],
        out_shape=[
            jax.ShapeDtypeStruct((B, L, H), jnp.float32),
            jax.ShapeDtypeStruct((B, L, A2), jnp.float32),
        ],
    )(x, w, am)


def _epilogue(x, acc, ses, sel, g, bb):
    recip = 1.0 / (ses[:, :NH] + 1e-16)
    mult = jnp.dot(recip, sel, preferred_element_type=jnp.float32)
    y = x + acc * mult
    m = jnp.mean(y, axis=-1, keepdims=True)
    v = jnp.mean((y - m) ** 2, axis=-1, keepdims=True)
    y = (y - m) / jnp.sqrt(v + 1e-5) * g + bb
    return jnp.maximum(y, 0.0)


def _post_dense_body(x_ref, acc_ref, ses_ref, sel_ref, g_ref, b_ref,
                     w_ref, am_ref, y_ref, h_ref, al_ref):
    y = _epilogue(x_ref[0], acc_ref[0], ses_ref[0], sel_ref[...],
                  g_ref[...], b_ref[...])
    y_ref[0] = y
    h = jnp.dot(y, w_ref[...], preferred_element_type=jnp.float32)
    h_ref[0] = h
    al_ref[0] = jnp.dot(h, am_ref[...], preferred_element_type=jnp.float32)


def _post_dense(x, acc, ses, sel, g, bb, w, am):
    return pl.pallas_call(
        _post_dense_body,
        grid=(B, L // _RT),
        in_specs=[
            pl.BlockSpec((1, _RT, H), lambda b, t: (b, t, 0)),
            pl.BlockSpec((1, _RT, H), lambda b, t: (b, t, 0)),
            pl.BlockSpec((1, _RT, SES), lambda b, t: (b, t, 0)),
            pl.BlockSpec((NH, H), lambda b, t: (0, 0)),
            pl.BlockSpec((1, H), lambda b, t: (0, 0)),
            pl.BlockSpec((1, H), lambda b, t: (0, 0)),
            pl.BlockSpec((H, H), lambda b, t: (0, 0)),
            pl.BlockSpec((H, A2), lambda b, t: (0, 0)),
        ],
        out_specs=[
            pl.BlockSpec((1, _RT, H), lambda b, t: (b, t, 0)),
            pl.BlockSpec((1, _RT, H), lambda b, t: (b, t, 0)),
            pl.BlockSpec((1, _RT, A2), lambda b, t: (b, t, 0)),
        ],
        out_shape=[
            jax.ShapeDtypeStruct((B, L, H), jnp.float32),
            jax.ShapeDtypeStruct((B, L, H), jnp.float32),
            jax.ShapeDtypeStruct((B, L, A2), jnp.float32),
        ],
    )(x, acc, ses, sel, g, bb, w, am)


def _post_mask_body(len_ref, x_ref, acc_ref, ses_ref, sel_ref, g_ref, b_ref,
                    o_ref):
    y = _epilogue(x_ref[0], acc_ref[0], ses_ref[0], sel_ref[...],
                  g_ref[...], b_ref[...])
    vl = len_ref[pl.program_id(0)]
    rows = (lax.broadcasted_iota(jnp.int32, (_RT, 1), 0)
            + pl.program_id(1) * _RT)
    o_ref[0] = jnp.where(rows < vl, y, 0.0)


def _post_mask(route_len, x, acc, ses, sel, g, bb):
    return pl.pallas_call(
        _post_mask_body,
        grid=(B, L // _RT),
        in_specs=[
            pl.BlockSpec(memory_space=pltpu.SMEM),
            pl.BlockSpec((1, _RT, H), lambda b, t: (b, t, 0)),
            pl.BlockSpec((1, _RT, H), lambda b, t: (b, t, 0)),
            pl.BlockSpec((1, _RT, SES), lambda b, t: (b, t, 0)),
            pl.BlockSpec((NH, H), lambda b, t: (0, 0)),
            pl.BlockSpec((1, H), lambda b, t: (0, 0)),
            pl.BlockSpec((1, H), lambda b, t: (0, 0)),
        ],
        out_specs=pl.BlockSpec((1, _RT, H), lambda b, t: (b, t, 0)),
        out_shape=jax.ShapeDtypeStruct((B, L, H), jnp.float32),
    )(route_len, x, acc, ses, sel, g, bb)


_MESH = plsc.VectorSubcoreMesh(core_axis_name="c", subcore_axis_name="s",
                               num_cores=NC)


@functools.partial(
    pl.kernel,
    mesh=_MESH,
    out_type=[
        jax.ShapeDtypeStruct((B * L, H), jnp.float32),
        jax.ShapeDtypeStruct((B * L, SES), jnp.float32),
    ],
    scratch_types=[
        pltpu.VMEM_SHARED((L, H), jnp.float32),
        pltpu.VMEM((L * A2,), jnp.float32),
        pltpu.VMEM((ZR, H), jnp.float32),
        pltpu.VMEM((CH, H), jnp.float32),
        pltpu.VMEM((CH, H), jnp.float32),
        pltpu.VMEM((EPT * NH,), jnp.float32),
        pltpu.VMEM((CH,), jnp.int32),
        pltpu.VMEM((CH,), jnp.int32),
        pltpu.VMEM((CH,), jnp.int32),
        pltpu.SemaphoreType.DMA,
    ],
    compiler_params=pltpu.CompilerParams(needs_layout_passes=False),
)
def _edge_kernel(h_hbm, al_hbm, ei_hbm, acc_hbm, ses_hbm,
                 acc_sp, alpha_buf, zrows, rows_buf, esrow_buf,
                 es_all, src_buf, dst_buf, gsrc_buf, sem):
    c = lax.axis_index("c")
    s = lax.axis_index("s")
    r0 = s * RPT
    zero16 = jnp.zeros((16,), jnp.float32)

    def _zrow(r, carry):
        for j in range(H // 16):
            zrows[r, pl.ds(j * 16, 16)] = zero16
        return carry
    lax.fori_loop(0, ZR, _zrow, 0)

    def _zesr(r, carry):
        for j in range(H // 16):
            esrow_buf[r, pl.ds(j * 16, 16)] = zero16
        return carry
    lax.fori_loop(0, CH, _zesr, 0)

    lane = lax.iota(jnp.int32, 16)
    lanehC = (lane & (NH - 1)) * CH
    mask4 = jnp.where(lane < NH, 1.0, 0.0).astype(jnp.float32)

    def _zero_acc():
        for z in range(RPT // ZR):
            pltpu.sync_copy(zrows, acc_sp.at[pl.ds(r0 + z * ZR, ZR)])

    def _graph(i, carry):
        b = c * GPC + i
        base = b * L
        pltpu.sync_copy(al_hbm.at[pl.ds(base * A2, L * A2)], alpha_buf)
        _zero_acc()
        plsc.subcore_barrier()

        def _chunk(j, carry2):
            e0 = s * EPT + j * CH
            pltpu.sync_copy(ei_hbm.at[2 * b, pl.ds(e0, CH)], src_buf)
            pltpu.sync_copy(ei_hbm.at[2 * b + 1, pl.ds(e0, CH)], dst_buf)
            ebase = j * NH * CH
            for g in range(CH // 16):
                gsrc_buf[pl.ds(g * 16, 16)] = (
                    src_buf[pl.ds(g * 16, 16)] + base)
            gcp = pltpu.async_copy(h_hbm.at[gsrc_buf], rows_buf, sem)
            for g in range(CH // 16):
                sv = src_buf[pl.ds(g * 16, 16)]
                dv = dst_buf[pl.ds(g * 16, 16)]
                s8 = sv * A2
                d8 = dv * A2
                for h in range(NH):
                    a_s = plsc.load_gather(alpha_buf, [s8 + h])
                    a_d = plsc.load_gather(alpha_buf, [d8 + (NH + h)])
                    es_all[pl.ds(ebase + h * CH + g * 16, 16)] = a_s + a_d
            for q in range(NH * CH // 16):
                sc = es_all[pl.ds(ebase + q * 16, 16)]
                es_all[pl.ds(ebase + q * 16, 16)] = jnp.exp(
                    jnp.maximum(sc, 0.2 * sc))
            gcp.wait()

            def _scale(k, carry3):
                eb = [plsc.load_gather(es_all,
                                       [jnp.full((16,), ebase + h * CH + k,
                                                 jnp.int32)])
                      for h in range(NH)]
                for j2 in range(H // 16):
                    rows_buf[k, pl.ds(j2 * 16, 16)] = (
                        rows_buf[k, pl.ds(j2 * 16, 16)] * eb[j2 // 2])
                return carry3
            lax.fori_loop(0, CH, _scale, 0)
            pltpu.sync_copy(rows_buf, acc_sp.at[dst_buf], add=True)
            return carry2
        lax.fori_loop(0, NCHUNK, _chunk, 0)
        plsc.subcore_barrier()
        pltpu.sync_copy(acc_sp.at[pl.ds(r0, RPT)],
                        acc_hbm.at[pl.ds(base + r0, RPT)])
        _zero_acc()
        plsc.subcore_barrier()

        def _chunk2(j, carry2):
            e0 = s * EPT + j * CH
            pltpu.sync_copy(ei_hbm.at[2 * b + 1, pl.ds(e0, CH)], dst_buf)

            def _fill(k, carry3):
                esr = plsc.load_gather(es_all, [lanehC + (k + j * NH * CH)])
                esrow_buf[k, pl.ds(0, 16)] = esr * mask4
                return carry3
            lax.fori_loop(0, CH, _fill, 0)
            pltpu.sync_copy(esrow_buf, acc_sp.at[dst_buf], add=True)
            return carry2
        lax.fori_loop(0, NCHUNK, _chunk2, 0)
        plsc.subcore_barrier()
        pltpu.sync_copy(acc_sp.at[pl.ds(r0, RPT)],
                        ses_hbm.at[pl.ds(base + r0, RPT)])
        return carry
    lax.fori_loop(0, GPC, _graph, 0)


def kernel(route_emb, route_len, adj_matrices, W0, a0, g0, b0, W1, a1, g1, b1):
    x0 = jnp.transpose(route_emb, (1, 0, 2))
    ei = adj_matrices.astype(jnp.int32).reshape(B * 2, E)
    am0 = _build_a_mat(a0)
    am1 = _build_a_mat(a1)
    sel = jnp.repeat(jnp.eye(NH, dtype=jnp.float32), HD, axis=1)
    g0r, b0r = g0.reshape(1, H), b0.reshape(1, H)
    g1r, b1r = g1.reshape(1, H), b1.reshape(1, H)

    h0, al0 = _dense(x0, W0, am0)
    acc0, ses0 = _edge_kernel(h0.reshape(B * L, H), al0.reshape(B * L * A2), ei)
    y1, h1, al1 = _post_dense(x0, acc0.reshape(B, L, H),
                              ses0.reshape(B, L, SES), sel, g0r, b0r, W1, am1)
    acc1, ses1 = _edge_kernel(h1.reshape(B * L, H), al1.reshape(B * L * A2), ei)
    out = _post_mask(route_len.astype(jnp.int32), y1, acc1.reshape(B, L, H),
                     ses1.reshape(B, L, SES), sel, g1r, b1r)
    return jnp.transpose(out, (1, 0, 2))


# speedup vs baseline: 46.5428x; 1.1843x over previous
import functools

import jax
import jax.numpy as jnp
from jax import lax
from jax.experimental import pallas as pl
from jax.experimental.pallas import tpu as pltpu
from jax.experimental.pallas import tpu_sc as plsc

L, B, H, E, NH = 4096, 8, 128, 16384, 4
HD = H // NH
A2 = 2 * NH
SES = 128

NC = 2
NT = 16
GPC = B // NC
EPT = E // NT
CH = 64
NCHUNK = EPT // CH
RPT = L // NT
ZR = 32

_RT = 512


def _build_a_mat(a):
    eye = jnp.eye(NH, dtype=jnp.float32)
    asrc = jnp.einsum('hd,hk->hdk', a[:, :HD], eye).reshape(H, NH)
    adst = jnp.einsum('hd,hk->hdk', a[:, HD:], eye).reshape(H, NH)
    return jnp.concatenate([asrc, adst], axis=1)


def _dense_body(x_ref, w_ref, am_ref, h_ref, al_ref):
    h = jnp.dot(x_ref[0], w_ref[...], preferred_element_type=jnp.float32)
    h_ref[0] = h
    al_ref[0] = jnp.dot(h, am_ref[...], preferred_element_type=jnp.float32)


def _dense(x, w, am):
    return pl.pallas_call(
        _dense_body,
        grid=(B, L // _RT),
        in_specs=[
            pl.BlockSpec((1, _RT, H), lambda b, t: (b, t, 0)),
            pl.BlockSpec((H, H), lambda b, t: (0, 0)),
            pl.BlockSpec((H, A2), lambda b, t: (0, 0)),
        ],
        out_specs=[
            pl.BlockSpec((1, _RT, H), lambda b, t: (b, t, 0)),
            pl.BlockSpec((1, _RT, A2), lambda b, t: (b, t, 0)),
        ],
        out_shape=[
            jax.ShapeDtypeStruct((B, L, H), jnp.float32),
            jax.ShapeDtypeStruct((B, L, A2), jnp.float32),
        ],
    )(x, w, am)


def _epilogue(x, acc, ses, sel, g, bb):
    recip = 1.0 / (ses[:, :NH] + 1e-16)
    mult = jnp.dot(recip, sel, preferred_element_type=jnp.float32)
    y = x + acc * mult
    m = jnp.mean(y, axis=-1, keepdims=True)
    v = jnp.mean((y - m) ** 2, axis=-1, keepdims=True)
    y = (y - m) / jnp.sqrt(v + 1e-5) * g + bb
    return jnp.maximum(y, 0.0)


def _post_dense_body(x_ref, acc_ref, ses_ref, sel_ref, g_ref, b_ref,
                     w_ref, am_ref, y_ref, h_ref, al_ref):
    y = _epilogue(x_ref[0], acc_ref[0], ses_ref[0], sel_ref[...],
                  g_ref[...], b_ref[...])
    y_ref[0] = y
    h = jnp.dot(y, w_ref[...], preferred_element_type=jnp.float32)
    h_ref[0] = h
    al_ref[0] = jnp.dot(h, am_ref[...], preferred_element_type=jnp.float32)


def _post_dense(x, acc, ses, sel, g, bb, w, am):
    return pl.pallas_call(
        _post_dense_body,
        grid=(B, L // _RT),
        in_specs=[
            pl.BlockSpec((1, _RT, H), lambda b, t: (b, t, 0)),
            pl.BlockSpec((1, _RT, H), lambda b, t: (b, t, 0)),
            pl.BlockSpec((1, _RT, SES), lambda b, t: (b, t, 0)),
            pl.BlockSpec((NH, H), lambda b, t: (0, 0)),
            pl.BlockSpec((1, H), lambda b, t: (0, 0)),
            pl.BlockSpec((1, H), lambda b, t: (0, 0)),
            pl.BlockSpec((H, H), lambda b, t: (0, 0)),
            pl.BlockSpec((H, A2), lambda b, t: (0, 0)),
        ],
        out_specs=[
            pl.BlockSpec((1, _RT, H), lambda b, t: (b, t, 0)),
            pl.BlockSpec((1, _RT, H), lambda b, t: (b, t, 0)),
            pl.BlockSpec((1, _RT, A2), lambda b, t: (b, t, 0)),
        ],
        out_shape=[
            jax.ShapeDtypeStruct((B, L, H), jnp.float32),
            jax.ShapeDtypeStruct((B, L, H), jnp.float32),
            jax.ShapeDtypeStruct((B, L, A2), jnp.float32),
        ],
    )(x, acc, ses, sel, g, bb, w, am)


def _post_mask_body(len_ref, x_ref, acc_ref, ses_ref, sel_ref, g_ref, b_ref,
                    o_ref):
    y = _epilogue(x_ref[0], acc_ref[0], ses_ref[0], sel_ref[...],
                  g_ref[...], b_ref[...])
    vl = len_ref[pl.program_id(0)]
    rows = (lax.broadcasted_iota(jnp.int32, (_RT, 1), 0)
            + pl.program_id(1) * _RT)
    o_ref[0] = jnp.where(rows < vl, y, 0.0)


def _post_mask(route_len, x, acc, ses, sel, g, bb):
    return pl.pallas_call(
        _post_mask_body,
        grid=(B, L // _RT),
        in_specs=[
            pl.BlockSpec(memory_space=pltpu.SMEM),
            pl.BlockSpec((1, _RT, H), lambda b, t: (b, t, 0)),
            pl.BlockSpec((1, _RT, H), lambda b, t: (b, t, 0)),
            pl.BlockSpec((1, _RT, SES), lambda b, t: (b, t, 0)),
            pl.BlockSpec((NH, H), lambda b, t: (0, 0)),
            pl.BlockSpec((1, H), lambda b, t: (0, 0)),
            pl.BlockSpec((1, H), lambda b, t: (0, 0)),
        ],
        out_specs=pl.BlockSpec((1, _RT, H), lambda b, t: (b, t, 0)),
        out_shape=jax.ShapeDtypeStruct((B, L, H), jnp.float32),
    )(route_len, x, acc, ses, sel, g, bb)


_MESH = plsc.VectorSubcoreMesh(core_axis_name="c", subcore_axis_name="s",
                               num_cores=NC)


@functools.partial(
    pl.kernel,
    mesh=_MESH,
    out_type=[
        jax.ShapeDtypeStruct((B * L, H), jnp.float32),
        jax.ShapeDtypeStruct((B * L, SES), jnp.float32),
    ],
    scratch_types=[
        pltpu.VMEM_SHARED((L, H), jnp.float32),
        pltpu.VMEM((L * A2,), jnp.float32),
        pltpu.VMEM((ZR, H), jnp.float32),
        pltpu.VMEM((CH, H), jnp.float32),
        pltpu.VMEM((CH, H), jnp.float32),
        pltpu.VMEM((EPT * NH,), jnp.float32),
        pltpu.VMEM((EPT,), jnp.int32),
        pltpu.VMEM((EPT,), jnp.int32),
        pltpu.VMEM((CH,), jnp.int32),
        pltpu.VMEM((CH,), jnp.int32),
        pltpu.SemaphoreType.DMA,
    ],
    compiler_params=pltpu.CompilerParams(needs_layout_passes=False),
)
def _edge_kernel(h_hbm, al_hbm, ei_hbm, acc_hbm, ses_hbm,
                 acc_sp, alpha_buf, zrows, rows_buf, esrow_buf,
                 es_all, src_all, dst_all, dst_buf, gsrc_buf, sem):
    c = lax.axis_index("c")
    s = lax.axis_index("s")
    r0 = s * RPT
    zero16 = jnp.zeros((16,), jnp.float32)

    def _zrow(r, carry):
        for j in range(H // 16):
            zrows[r, pl.ds(j * 16, 16)] = zero16
        return carry
    lax.fori_loop(0, ZR, _zrow, 0)

    def _zesr(r, carry):
        for j in range(H // 16):
            esrow_buf[r, pl.ds(j * 16, 16)] = zero16
        return carry
    lax.fori_loop(0, CH, _zesr, 0)

    lane = lax.iota(jnp.int32, 16)
    lanehC = (lane & (NH - 1)) * CH
    mask4 = jnp.where(lane < NH, 1.0, 0.0).astype(jnp.float32)

    def _zero_acc():
        for z in range(RPT // ZR):
            pltpu.sync_copy(zrows, acc_sp.at[pl.ds(r0 + z * ZR, ZR)])

    def _graph(i, carry):
        b = c * GPC + i
        base = b * L
        pltpu.sync_copy(al_hbm.at[pl.ds(base * A2, L * A2)], alpha_buf)
        pltpu.sync_copy(ei_hbm.at[2 * b, pl.ds(s * EPT, EPT)], src_all)
        pltpu.sync_copy(ei_hbm.at[2 * b + 1, pl.ds(s * EPT, EPT)], dst_all)
        _zero_acc()
        plsc.subcore_barrier()

        def _chunk(j, carry2):
            c0 = j * CH
            ebase = j * NH * CH
            for g in range(CH // 16):
                gsrc_buf[pl.ds(g * 16, 16)] = (
                    src_all[pl.ds(c0 + g * 16, 16)] + base)
                dst_buf[pl.ds(g * 16, 16)] = dst_all[pl.ds(c0 + g * 16, 16)]
            gcp = pltpu.async_copy(h_hbm.at[gsrc_buf], rows_buf, sem)
            for g in range(CH // 16):
                sv = src_all[pl.ds(c0 + g * 16, 16)]
                dv = dst_all[pl.ds(c0 + g * 16, 16)]
                s8 = sv * A2
                d8 = dv * A2
                for h in range(NH):
                    a_s = plsc.load_gather(alpha_buf, [s8 + h])
                    a_d = plsc.load_gather(alpha_buf, [d8 + (NH + h)])
                    es_all[pl.ds(ebase + h * CH + g * 16, 16)] = a_s + a_d
            for q in range(NH * CH // 16):
                sc = es_all[pl.ds(ebase + q * 16, 16)]
                es_all[pl.ds(ebase + q * 16, 16)] = jnp.exp(
                    jnp.maximum(sc, 0.2 * sc))
            gcp.wait()

            def _scale(k, carry3):
                eb = [plsc.load_gather(es_all,
                                       [jnp.full((16,), ebase + h * CH + k,
                                                 jnp.int32)])
                      for h in range(NH)]
                for j2 in range(H // 16):
                    rows_buf[k, pl.ds(j2 * 16, 16)] = (
                        rows_buf[k, pl.ds(j2 * 16, 16)] * eb[j2 // 2])
                return carry3
            lax.fori_loop(0, CH, _scale, 0)
            pltpu.sync_copy(rows_buf, acc_sp.at[dst_buf], add=True)
            return carry2
        lax.fori_loop(0, NCHUNK, _chunk, 0)
        plsc.subcore_barrier()
        pltpu.sync_copy(acc_sp.at[pl.ds(r0, RPT)],
                        acc_hbm.at[pl.ds(base + r0, RPT)])
        _zero_acc()
        plsc.subcore_barrier()

        def _chunk2(j, carry2):
            c0 = j * CH
            for g in range(CH // 16):
                dst_buf[pl.ds(g * 16, 16)] = dst_all[pl.ds(c0 + g * 16, 16)]

            def _fill(k, carry3):
                esr = plsc.load_gather(es_all, [lanehC + (k + j * NH * CH)])
                esrow_buf[k, pl.ds(0, 16)] = esr * mask4
                return carry3
            lax.fori_loop(0, CH, _fill, 0)
            pltpu.sync_copy(esrow_buf, acc_sp.at[dst_buf], add=True)
            return carry2
        lax.fori_loop(0, NCHUNK, _chunk2, 0)
        plsc.subcore_barrier()
        pltpu.sync_copy(acc_sp.at[pl.ds(r0, RPT)],
                        ses_hbm.at[pl.ds(base + r0, RPT)])
        return carry
    lax.fori_loop(0, GPC, _graph, 0)


def kernel(route_emb, route_len, adj_matrices, W0, a0, g0, b0, W1, a1, g1, b1):
    x0 = jnp.transpose(route_emb, (1, 0, 2))
    ei = adj_matrices.astype(jnp.int32).reshape(B * 2, E)
    am0 = _build_a_mat(a0)
    am1 = _build_a_mat(a1)
    sel = jnp.repeat(jnp.eye(NH, dtype=jnp.float32), HD, axis=1)
    g0r, b0r = g0.reshape(1, H), b0.reshape(1, H)
    g1r, b1r = g1.reshape(1, H), b1.reshape(1, H)

    h0, al0 = _dense(x0, W0, am0)
    acc0, ses0 = _edge_kernel(h0.reshape(B * L, H), al0.reshape(B * L * A2), ei)
    y1, h1, al1 = _post_dense(x0, acc0.reshape(B, L, H),
                              ses0.reshape(B, L, SES), sel, g0r, b0r, W1, am1)
    acc1, ses1 = _edge_kernel(h1.reshape(B * L, H), al1.reshape(B * L * A2), ei)
    out = _post_mask(route_len.astype(jnp.int32), y1, acc1.reshape(B, L, H),
                     ses1.reshape(B, L, SES), sel, g1r, b1r)
    return jnp.transpose(out, (1, 0, 2))


# speedup vs baseline: 46.6876x; 1.0031x over previous
import functools

import jax
import jax.numpy as jnp
from jax import lax
from jax.experimental import pallas as pl
from jax.experimental.pallas import tpu as pltpu
from jax.experimental.pallas import tpu_sc as plsc

L, B, H, E, NH = 4096, 8, 128, 16384, 4
HD = H // NH
A2 = 2 * NH
SES = 128

NC = 2
NT = 16
GPC = B // NC
EPT = E // NT
CH = 64
NCHUNK = EPT // CH
RPT = L // NT
ZR = 32

_RT = 512


def _build_a_mat(a):
    eye = jnp.eye(NH, dtype=jnp.float32)
    asrc = jnp.einsum('hd,hk->hdk', a[:, :HD], eye).reshape(H, NH)
    adst = jnp.einsum('hd,hk->hdk', a[:, HD:], eye).reshape(H, NH)
    return jnp.concatenate([asrc, adst], axis=1)


def _dense_body(x_ref, w_ref, am_ref, h_ref, al_ref):
    h = jnp.dot(x_ref[0], w_ref[...], preferred_element_type=jnp.float32)
    h_ref[0] = h
    al_ref[0] = jnp.dot(h, am_ref[...], preferred_element_type=jnp.float32)


def _dense(x, w, am):
    return pl.pallas_call(
        _dense_body,
        grid=(B, L // _RT),
        in_specs=[
            pl.BlockSpec((1, _RT, H), lambda b, t: (b, t, 0)),
            pl.BlockSpec((H, H), lambda b, t: (0, 0)),
            pl.BlockSpec((H, A2), lambda b, t: (0, 0)),
        ],
        out_specs=[
            pl.BlockSpec((1, _RT, H), lambda b, t: (b, t, 0)),
            pl.BlockSpec((1, _RT, A2), lambda b, t: (b, t, 0)),
        ],
        out_shape=[
            jax.ShapeDtypeStruct((B, L, H), jnp.float32),
            jax.ShapeDtypeStruct((B, L, A2), jnp.float32),
        ],
    )(x, w, am)


def _epilogue(x, acc, ses, sel, g, bb):
    recip = 1.0 / (ses[:, :NH] + 1e-16)
    mult = jnp.dot(recip, sel, preferred_element_type=jnp.float32)
    y = x + acc * mult
    m = jnp.mean(y, axis=-1, keepdims=True)
    v = jnp.mean((y - m) ** 2, axis=-1, keepdims=True)
    y = (y - m) / jnp.sqrt(v + 1e-5) * g + bb
    return jnp.maximum(y, 0.0)


def _post_dense_body(x_ref, acc_ref, ses_ref, sel_ref, g_ref, b_ref,
                     w_ref, am_ref, y_ref, h_ref, al_ref):
    y = _epilogue(x_ref[0], acc_ref[0], ses_ref[0], sel_ref[...],
                  g_ref[...], b_ref[...])
    y_ref[0] = y
    h = jnp.dot(y, w_ref[...], preferred_element_type=jnp.float32)
    h_ref[0] = h
    al_ref[0] = jnp.dot(h, am_ref[...], preferred_element_type=jnp.float32)


def _post_dense(x, acc, ses, sel, g, bb, w, am):
    return pl.pallas_call(
        _post_dense_body,
        grid=(B, L // _RT),
        in_specs=[
            pl.BlockSpec((1, _RT, H), lambda b, t: (b, t, 0)),
            pl.BlockSpec((1, _RT, H), lambda b, t: (b, t, 0)),
            pl.BlockSpec((1, _RT, SES), lambda b, t: (b, t, 0)),
            pl.BlockSpec((NH, H), lambda b, t: (0, 0)),
            pl.BlockSpec((1, H), lambda b, t: (0, 0)),
            pl.BlockSpec((1, H), lambda b, t: (0, 0)),
            pl.BlockSpec((H, H), lambda b, t: (0, 0)),
            pl.BlockSpec((H, A2), lambda b, t: (0, 0)),
        ],
        out_specs=[
            pl.BlockSpec((1, _RT, H), lambda b, t: (b, t, 0)),
            pl.BlockSpec((1, _RT, H), lambda b, t: (b, t, 0)),
            pl.BlockSpec((1, _RT, A2), lambda b, t: (b, t, 0)),
        ],
        out_shape=[
            jax.ShapeDtypeStruct((B, L, H), jnp.float32),
            jax.ShapeDtypeStruct((B, L, H), jnp.float32),
            jax.ShapeDtypeStruct((B, L, A2), jnp.float32),
        ],
    )(x, acc, ses, sel, g, bb, w, am)


def _post_mask_body(len_ref, x_ref, acc_ref, ses_ref, sel_ref, g_ref, b_ref,
                    o_ref):
    y = _epilogue(x_ref[0], acc_ref[0], ses_ref[0], sel_ref[...],
                  g_ref[...], b_ref[...])
    vl = len_ref[pl.program_id(0)]
    rows = (lax.broadcasted_iota(jnp.int32, (_RT, 1), 0)
            + pl.program_id(1) * _RT)
    o_ref[0] = jnp.where(rows < vl, y, 0.0)


def _post_mask(route_len, x, acc, ses, sel, g, bb):
    return pl.pallas_call(
        _post_mask_body,
        grid=(B, L // _RT),
        in_specs=[
            pl.BlockSpec(memory_space=pltpu.SMEM),
            pl.BlockSpec((1, _RT, H), lambda b, t: (b, t, 0)),
            pl.BlockSpec((1, _RT, H), lambda b, t: (b, t, 0)),
            pl.BlockSpec((1, _RT, SES), lambda b, t: (b, t, 0)),
            pl.BlockSpec((NH, H), lambda b, t: (0, 0)),
            pl.BlockSpec((1, H), lambda b, t: (0, 0)),
            pl.BlockSpec((1, H), lambda b, t: (0, 0)),
        ],
        out_specs=pl.BlockSpec((1, _RT, H), lambda b, t: (b, t, 0)),
        out_shape=jax.ShapeDtypeStruct((B, L, H), jnp.float32),
    )(route_len, x, acc, ses, sel, g, bb)


_MESH = plsc.VectorSubcoreMesh(core_axis_name="c", subcore_axis_name="s",
                               num_cores=NC)


@functools.partial(
    pl.kernel,
    mesh=_MESH,
    out_type=[
        jax.ShapeDtypeStruct((B * L, H), jnp.float32),
        jax.ShapeDtypeStruct((B * L, SES), jnp.float32),
    ],
    scratch_types=[
        pltpu.VMEM_SHARED((L, H), jnp.float32),
        pltpu.VMEM((L * A2,), jnp.float32),
        pltpu.VMEM((CH, H), jnp.float32),
        pltpu.VMEM((CH, H), jnp.float32),
        pltpu.VMEM((CH, H), jnp.float32),
        pltpu.VMEM((EPT * NH,), jnp.float32),
        pltpu.VMEM((EPT,), jnp.int32),
        pltpu.VMEM((EPT,), jnp.int32),
        pltpu.VMEM((CH,), jnp.int32),
        pltpu.VMEM((CH,), jnp.int32),
        pltpu.VMEM((CH,), jnp.int32),
        pltpu.SemaphoreType.DMA,
        pltpu.SemaphoreType.DMA,
        pltpu.SemaphoreType.DMA,
    ],
    compiler_params=pltpu.CompilerParams(needs_layout_passes=False),
)
def _edge_kernel(h_hbm, al_hbm, ei_hbm, z_hbm, acc_hbm, ses_hbm,
                 acc_sp, alpha_buf, rows_a, rows_b, esrow_buf,
                 es_all, src_all, dst_all, dst_a, dst_b, gsrc_buf,
                 semg, sema, semb):
    c = lax.axis_index("c")
    s = lax.axis_index("s")
    r0 = s * RPT
    zero16 = jnp.zeros((16,), jnp.float32)

    def _zesr(r, carry):
        for j in range(H // 16):
            esrow_buf[r, pl.ds(j * 16, 16)] = zero16
        return carry
    lax.fori_loop(0, CH, _zesr, 0)

    lane = lax.iota(jnp.int32, 16)
    lanehC = (lane & (NH - 1)) * CH
    mask4 = jnp.where(lane < NH, 1.0, 0.0).astype(jnp.float32)

    def _zero_acc():
        pltpu.sync_copy(z_hbm, acc_sp.at[pl.ds(r0, RPT)])

    def _graph(i, carry):
        b = c * GPC + i
        base = b * L
        pltpu.sync_copy(al_hbm.at[pl.ds(base * A2, L * A2)], alpha_buf)
        pltpu.sync_copy(ei_hbm.at[2 * b, pl.ds(s * EPT, EPT)], src_all)
        pltpu.sync_copy(ei_hbm.at[2 * b + 1, pl.ds(s * EPT, EPT)], dst_all)
        _zero_acc()
        plsc.subcore_barrier()

        def _half_chunk(j, rows_buf, dst_buf, sems, first):
            c0 = j * CH
            ebase = j * NH * CH
            if not first:
                pltpu.make_async_copy(h_hbm.at[pl.ds(0, CH)], rows_buf,
                                      sems).wait()
            for g in range(CH // 16):
                gsrc_buf[pl.ds(g * 16, 16)] = (
                    src_all[pl.ds(c0 + g * 16, 16)] + base)
                dst_buf[pl.ds(g * 16, 16)] = dst_all[pl.ds(c0 + g * 16, 16)]
            gcp = pltpu.async_copy(h_hbm.at[gsrc_buf], rows_buf, semg)
            for g in range(CH // 16):
                sv = src_all[pl.ds(c0 + g * 16, 16)]
                dv = dst_all[pl.ds(c0 + g * 16, 16)]
                s8 = sv * A2
                d8 = dv * A2
                for h in range(NH):
                    a_s = plsc.load_gather(alpha_buf, [s8 + h])
                    a_d = plsc.load_gather(alpha_buf, [d8 + (NH + h)])
                    es_all[pl.ds(ebase + h * CH + g * 16, 16)] = a_s + a_d
            for q in range(NH * CH // 16):
                sc = es_all[pl.ds(ebase + q * 16, 16)]
                es_all[pl.ds(ebase + q * 16, 16)] = jnp.exp(
                    jnp.maximum(sc, 0.2 * sc))
            gcp.wait()

            def _scale(k, carry3):
                eb = [plsc.load_gather(es_all,
                                       [jnp.full((16,), ebase + h * CH + k,
                                                 jnp.int32)])
                      for h in range(NH)]
                for j2 in range(H // 16):
                    rows_buf[k, pl.ds(j2 * 16, 16)] = (
                        rows_buf[k, pl.ds(j2 * 16, 16)] * eb[j2 // 2])
                return carry3
            lax.fori_loop(0, CH, _scale, 0)
            pltpu.async_copy(rows_buf, acc_sp.at[dst_buf], sems, add=True)

        def _chunk_pair(t, carry2):
            @pl.when(t == 0)
            def _():
                _half_chunk(2 * t, rows_a, dst_a, sema, True)
                _half_chunk(2 * t + 1, rows_b, dst_b, semb, True)

            @pl.when(t > 0)
            def _():
                _half_chunk(2 * t, rows_a, dst_a, sema, False)
                _half_chunk(2 * t + 1, rows_b, dst_b, semb, False)
            return carry2
        lax.fori_loop(0, NCHUNK // 2, _chunk_pair, 0)
        pltpu.make_async_copy(h_hbm.at[pl.ds(0, CH)], rows_a, sema).wait()
        pltpu.make_async_copy(h_hbm.at[pl.ds(0, CH)], rows_b, semb).wait()
        plsc.subcore_barrier()
        pltpu.sync_copy(acc_sp.at[pl.ds(r0, RPT)],
                        acc_hbm.at[pl.ds(base + r0, RPT)])
        _zero_acc()
        plsc.subcore_barrier()

        def _chunk2(j, carry2):
            c0 = j * CH
            for g in range(CH // 16):
                dst_a[pl.ds(g * 16, 16)] = dst_all[pl.ds(c0 + g * 16, 16)]

            def _fill(k, carry3):
                esr = plsc.load_gather(es_all, [lanehC + (k + j * NH * CH)])
                esrow_buf[k, pl.ds(0, 16)] = esr * mask4
                return carry3
            lax.fori_loop(0, CH, _fill, 0)
            pltpu.sync_copy(esrow_buf, acc_sp.at[dst_a], add=True)
            return carry2
        lax.fori_loop(0, NCHUNK, _chunk2, 0)
        plsc.subcore_barrier()
        pltpu.sync_copy(acc_sp.at[pl.ds(r0, RPT)],
                        ses_hbm.at[pl.ds(base + r0, RPT)])
        return carry
    lax.fori_loop(0, GPC, _graph, 0)


def kernel(route_emb, route_len, adj_matrices, W0, a0, g0, b0, W1, a1, g1, b1):
    x0 = jnp.transpose(route_emb, (1, 0, 2))
    ei = adj_matrices.astype(jnp.int32).reshape(B * 2, E)
    am0 = _build_a_mat(a0)
    am1 = _build_a_mat(a1)
    sel = jnp.repeat(jnp.eye(NH, dtype=jnp.float32), HD, axis=1)
    g0r, b0r = g0.reshape(1, H), b0.reshape(1, H)
    g1r, b1r = g1.reshape(1, H), b1.reshape(1, H)

    zblk = jnp.zeros((RPT, H), jnp.float32)
    h0, al0 = _dense(x0, W0, am0)
    acc0, ses0 = _edge_kernel(h0.reshape(B * L, H), al0.reshape(B * L * A2),
                              ei, zblk)
    y1, h1, al1 = _post_dense(x0, acc0.reshape(B, L, H),
                              ses0.reshape(B, L, SES), sel, g0r, b0r, W1, am1)
    acc1, ses1 = _edge_kernel(h1.reshape(B * L, H), al1.reshape(B * L * A2),
                              ei, zblk)
    out = _post_mask(route_len.astype(jnp.int32), y1, acc1.reshape(B, L, H),
                     ses1.reshape(B, L, SES), sel, g1r, b1r)
    return jnp.transpose(out, (1, 0, 2))


# speedup vs baseline: 51.7468x; 1.1084x over previous
import functools

import jax
import jax.numpy as jnp
from jax import lax
from jax.experimental import pallas as pl
from jax.experimental.pallas import tpu as pltpu
from jax.experimental.pallas import tpu_sc as plsc

L, B, H, E, NH = 4096, 8, 128, 16384, 4
HD = H // NH
A2 = 2 * NH
SES = 128

NC = 2
NT = 16
GPC = B // NC
EPT = E // NT
CH = 64
NCHUNK = EPT // CH
RPT = L // NT
ZR = 32

_RT = 512


def _build_a_mat(a):
    eye = jnp.eye(NH, dtype=jnp.float32)
    asrc = jnp.einsum('hd,hk->hdk', a[:, :HD], eye).reshape(H, NH)
    adst = jnp.einsum('hd,hk->hdk', a[:, HD:], eye).reshape(H, NH)
    return jnp.concatenate([asrc, adst], axis=1)


def _dense_body(x_ref, w_ref, am_ref, h_ref, al_ref):
    h = jnp.dot(x_ref[0], w_ref[...], preferred_element_type=jnp.float32)
    h_ref[0] = h
    al_ref[0] = jnp.dot(h, am_ref[...], preferred_element_type=jnp.float32)


def _dense(x, w, am):
    return pl.pallas_call(
        _dense_body,
        grid=(B, L // _RT),
        in_specs=[
            pl.BlockSpec((1, _RT, H), lambda b, t: (b, t, 0)),
            pl.BlockSpec((H, H), lambda b, t: (0, 0)),
            pl.BlockSpec((H, A2), lambda b, t: (0, 0)),
        ],
        out_specs=[
            pl.BlockSpec((1, _RT, H), lambda b, t: (b, t, 0)),
            pl.BlockSpec((1, _RT, A2), lambda b, t: (b, t, 0)),
        ],
        out_shape=[
            jax.ShapeDtypeStruct((B, L, H), jnp.float32),
            jax.ShapeDtypeStruct((B, L, A2), jnp.float32),
        ],
    )(x, w, am)


def _epilogue(x, acc, ses, sel, g, bb):
    recip = 1.0 / (ses[:, :NH] + 1e-16)
    mult = jnp.dot(recip, sel, preferred_element_type=jnp.float32)
    y = x + acc * mult
    m = jnp.mean(y, axis=-1, keepdims=True)
    v = jnp.mean((y - m) ** 2, axis=-1, keepdims=True)
    y = (y - m) / jnp.sqrt(v + 1e-5) * g + bb
    return jnp.maximum(y, 0.0)


def _post_dense_body(x_ref, acc_ref, ses_ref, sel_ref, g_ref, b_ref,
                     w_ref, am_ref, y_ref, h_ref, al_ref):
    y = _epilogue(x_ref[0], acc_ref[0], ses_ref[0], sel_ref[...],
                  g_ref[...], b_ref[...])
    y_ref[0] = y
    h = jnp.dot(y, w_ref[...], preferred_element_type=jnp.float32)
    h_ref[0] = h
    al_ref[0] = jnp.dot(h, am_ref[...], preferred_element_type=jnp.float32)


def _post_dense(x, acc, ses, sel, g, bb, w, am):
    return pl.pallas_call(
        _post_dense_body,
        grid=(B, L // _RT),
        in_specs=[
            pl.BlockSpec((1, _RT, H), lambda b, t: (b, t, 0)),
            pl.BlockSpec((1, _RT, H), lambda b, t: (b, t, 0)),
            pl.BlockSpec((1, _RT, SES), lambda b, t: (b, t, 0)),
            pl.BlockSpec((NH, H), lambda b, t: (0, 0)),
            pl.BlockSpec((1, H), lambda b, t: (0, 0)),
            pl.BlockSpec((1, H), lambda b, t: (0, 0)),
            pl.BlockSpec((H, H), lambda b, t: (0, 0)),
            pl.BlockSpec((H, A2), lambda b, t: (0, 0)),
        ],
        out_specs=[
            pl.BlockSpec((1, _RT, H), lambda b, t: (b, t, 0)),
            pl.BlockSpec((1, _RT, H), lambda b, t: (b, t, 0)),
            pl.BlockSpec((1, _RT, A2), lambda b, t: (b, t, 0)),
        ],
        out_shape=[
            jax.ShapeDtypeStruct((B, L, H), jnp.float32),
            jax.ShapeDtypeStruct((B, L, H), jnp.float32),
            jax.ShapeDtypeStruct((B, L, A2), jnp.float32),
        ],
    )(x, acc, ses, sel, g, bb, w, am)


def _post_mask_body(len_ref, x_ref, acc_ref, ses_ref, sel_ref, g_ref, b_ref,
                    o_ref):
    y = _epilogue(x_ref[0], acc_ref[0], ses_ref[0], sel_ref[...],
                  g_ref[...], b_ref[...])
    vl = len_ref[pl.program_id(0)]
    rows = (lax.broadcasted_iota(jnp.int32, (_RT, 1), 0)
            + pl.program_id(1) * _RT)
    o_ref[0] = jnp.where(rows < vl, y, 0.0)


def _post_mask(route_len, x, acc, ses, sel, g, bb):
    return pl.pallas_call(
        _post_mask_body,
        grid=(B, L // _RT),
        in_specs=[
            pl.BlockSpec(memory_space=pltpu.SMEM),
            pl.BlockSpec((1, _RT, H), lambda b, t: (b, t, 0)),
            pl.BlockSpec((1, _RT, H), lambda b, t: (b, t, 0)),
            pl.BlockSpec((1, _RT, SES), lambda b, t: (b, t, 0)),
            pl.BlockSpec((NH, H), lambda b, t: (0, 0)),
            pl.BlockSpec((1, H), lambda b, t: (0, 0)),
            pl.BlockSpec((1, H), lambda b, t: (0, 0)),
        ],
        out_specs=pl.BlockSpec((1, _RT, H), lambda b, t: (b, t, 0)),
        out_shape=jax.ShapeDtypeStruct((B, L, H), jnp.float32),
    )(route_len, x, acc, ses, sel, g, bb)


_MESH = plsc.VectorSubcoreMesh(core_axis_name="c", subcore_axis_name="s",
                               num_cores=NC)


@functools.partial(
    pl.kernel,
    mesh=_MESH,
    out_type=[
        jax.ShapeDtypeStruct((B * L, H), jnp.float32),
        jax.ShapeDtypeStruct((B * L, SES), jnp.float32),
    ],
    scratch_types=[
        pltpu.VMEM_SHARED((L, H), jnp.float32),
        pltpu.VMEM((L * A2,), jnp.float32),
        pltpu.VMEM((CH, H), jnp.float32),
        pltpu.VMEM((CH, H), jnp.float32),
        pltpu.VMEM((CH, H), jnp.float32),
        pltpu.VMEM((EPT * NH,), jnp.float32),
        pltpu.VMEM((EPT,), jnp.int32),
        pltpu.VMEM((EPT,), jnp.int32),
        pltpu.VMEM((CH,), jnp.int32),
        pltpu.VMEM((CH,), jnp.int32),
        pltpu.VMEM((CH,), jnp.int32),
        pltpu.SemaphoreType.DMA,
        pltpu.SemaphoreType.DMA,
        pltpu.SemaphoreType.DMA,
    ],
    compiler_params=pltpu.CompilerParams(needs_layout_passes=False),
)
def _edge_kernel(h_hbm, al_hbm, ei_hbm, z_hbm, acc_hbm, ses_hbm,
                 acc_sp, alpha_buf, rows_a, rows_b, esrow_buf,
                 es_all, src_all, dst_all, dst_a, dst_b, gsrc_buf,
                 semg, sema, semb):
    c = lax.axis_index("c")
    s = lax.axis_index("s")
    r0 = s * RPT
    zero16 = jnp.zeros((16,), jnp.float32)

    def _zesr(r, carry):
        for j in range(H // 16):
            esrow_buf[r, pl.ds(j * 16, 16)] = zero16
        return carry
    lax.fori_loop(0, CH, _zesr, 0)

    lane = lax.iota(jnp.int32, 16)
    lanehC = (lane & (NH - 1)) * CH
    mask4 = jnp.where(lane < NH, 1.0, 0.0).astype(jnp.float32)

    def _zero_acc():
        pltpu.sync_copy(z_hbm, acc_sp.at[pl.ds(r0, RPT)])

    def _graph(i, carry):
        b = c * GPC + i
        base = b * L
        pltpu.sync_copy(al_hbm.at[pl.ds(base * A2, L * A2)], alpha_buf)
        pltpu.sync_copy(ei_hbm.at[2 * b, pl.ds(s * EPT, EPT)], src_all)
        pltpu.sync_copy(ei_hbm.at[2 * b + 1, pl.ds(s * EPT, EPT)], dst_all)
        _zero_acc()
        plsc.subcore_barrier()

        def _half_chunk(j, rows_buf, dst_buf, sems, first):
            c0 = j * CH
            ebase = j * NH * CH
            if not first:
                pltpu.make_async_copy(h_hbm.at[pl.ds(0, CH)], rows_buf,
                                      sems).wait()
            for g in range(CH // 16):
                gsrc_buf[pl.ds(g * 16, 16)] = (
                    src_all[pl.ds(c0 + g * 16, 16)] + base)
                dst_buf[pl.ds(g * 16, 16)] = dst_all[pl.ds(c0 + g * 16, 16)]
            gcp = pltpu.async_copy(h_hbm.at[gsrc_buf], rows_buf, semg)
            for g in range(CH // 16):
                sv = src_all[pl.ds(c0 + g * 16, 16)]
                dv = dst_all[pl.ds(c0 + g * 16, 16)]
                s8 = sv * A2
                d8 = dv * A2
                for h in range(NH):
                    a_s = plsc.load_gather(alpha_buf, [s8 + h])
                    a_d = plsc.load_gather(alpha_buf, [d8 + (NH + h)])
                    es_all[pl.ds(ebase + h * CH + g * 16, 16)] = a_s + a_d
            for q in range(NH * CH // 16):
                sc = es_all[pl.ds(ebase + q * 16, 16)]
                es_all[pl.ds(ebase + q * 16, 16)] = jnp.exp(
                    jnp.maximum(sc, 0.2 * sc))
            gcp.wait()

            @plsc.parallel_loop(0, CH, unroll=2)
            def _scale(k):
                eb = [plsc.load_gather(es_all,
                                       [jnp.full((16,), ebase + h * CH + k,
                                                 jnp.int32)])
                      for h in range(NH)]
                for j2 in range(H // 16):
                    rows_buf[k, pl.ds(j2 * 16, 16)] = (
                        rows_buf[k, pl.ds(j2 * 16, 16)] * eb[j2 // 2])
            pltpu.async_copy(rows_buf, acc_sp.at[dst_buf], sems, add=True)

        def _chunk_pair(t, carry2):
            @pl.when(t == 0)
            def _():
                _half_chunk(2 * t, rows_a, dst_a, sema, True)
                _half_chunk(2 * t + 1, rows_b, dst_b, semb, True)

            @pl.when(t > 0)
            def _():
                _half_chunk(2 * t, rows_a, dst_a, sema, False)
                _half_chunk(2 * t + 1, rows_b, dst_b, semb, False)
            return carry2
        lax.fori_loop(0, NCHUNK // 2, _chunk_pair, 0)
        pltpu.make_async_copy(h_hbm.at[pl.ds(0, CH)], rows_a, sema).wait()
        pltpu.make_async_copy(h_hbm.at[pl.ds(0, CH)], rows_b, semb).wait()
        plsc.subcore_barrier()
        pltpu.sync_copy(acc_sp.at[pl.ds(r0, RPT)],
                        acc_hbm.at[pl.ds(base + r0, RPT)])
        _zero_acc()
        plsc.subcore_barrier()

        def _chunk2(j, carry2):
            c0 = j * CH
            for g in range(CH // 16):
                dst_a[pl.ds(g * 16, 16)] = dst_all[pl.ds(c0 + g * 16, 16)]

            @plsc.parallel_loop(0, CH, unroll=4)
            def _fill(k):
                esr = plsc.load_gather(es_all, [lanehC + (k + j * NH * CH)])
                esrow_buf[k, pl.ds(0, 16)] = esr * mask4
            pltpu.sync_copy(esrow_buf, acc_sp.at[dst_a], add=True)
            return carry2
        lax.fori_loop(0, NCHUNK, _chunk2, 0)
        plsc.subcore_barrier()
        pltpu.sync_copy(acc_sp.at[pl.ds(r0, RPT)],
                        ses_hbm.at[pl.ds(base + r0, RPT)])
        return carry
    lax.fori_loop(0, GPC, _graph, 0)


def kernel(route_emb, route_len, adj_matrices, W0, a0, g0, b0, W1, a1, g1, b1):
    x0 = jnp.transpose(route_emb, (1, 0, 2))
    ei = adj_matrices.astype(jnp.int32).reshape(B * 2, E)
    am0 = _build_a_mat(a0)
    am1 = _build_a_mat(a1)
    sel = jnp.repeat(jnp.eye(NH, dtype=jnp.float32), HD, axis=1)
    g0r, b0r = g0.reshape(1, H), b0.reshape(1, H)
    g1r, b1r = g1.reshape(1, H), b1.reshape(1, H)

    zblk = jnp.zeros((RPT, H), jnp.float32)
    h0, al0 = _dense(x0, W0, am0)
    acc0, ses0 = _edge_kernel(h0.reshape(B * L, H), al0.reshape(B * L * A2),
                              ei, zblk)
    y1, h1, al1 = _post_dense(x0, acc0.reshape(B, L, H),
                              ses0.reshape(B, L, SES), sel, g0r, b0r, W1, am1)
    acc1, ses1 = _edge_kernel(h1.reshape(B * L, H), al1.reshape(B * L * A2),
                              ei, zblk)
    out = _post_mask(route_len.astype(jnp.int32), y1, acc1.reshape(B, L, H),
                     ses1.reshape(B, L, SES), sel, g1r, b1r)
    return jnp.transpose(out, (1, 0, 2))
